# baseline (speedup 1.0000x reference)
import jax
import jax.numpy as jnp
from jax.experimental import pallas as pl
from jax.experimental.pallas import tpu as pltpu

B, L, H = 16, 2048, 1536
E = 256
C = 3
NC = 26

CH = 2048
NBUF = 4
LAG = 2


def _make_body(nb):
    rows_n = nb * L
    nch = rows_n // CH

    def _body(hid_ref, pos_ref, ids_ref, msk_ref, table_ref, w_ref, bias_ref,
              out_ref, buf_ref, rows_ref, in_sems, out_sems, row_sem):
        def in_cp(k):
            return pltpu.make_async_copy(
                hid_ref.at[pl.ds(k * CH, CH)], buf_ref.at[k % NBUF],
                in_sems.at[k % NBUF])

        def out_cp(k):
            return pltpu.make_async_copy(
                buf_ref.at[k % NBUF], out_ref.at[pl.ds(k * CH, CH)],
                out_sems.at[k % NBUF])

        for k in range(NBUF):
            in_cp(k).start()

        gathers = []
        for i in range(nb):
            g = i * L + pos_ref[i]
            cp = pltpu.make_async_copy(hid_ref.at[g], rows_ref.at[i], row_sem)
            cp.start()
            gathers.append(cp)

        ids = ids_ref[...]
        msk = msk_ref[...]
        denom = jnp.maximum(jnp.sum(msk, axis=1, keepdims=True), 1.0)
        wcoef = msk / denom
        iota_n = jax.lax.broadcasted_iota(jnp.int32, (nb, NC), 1)
        wsel = jnp.zeros((nb, NC), jnp.float32)
        for c in range(C):
            wsel = wsel + jnp.where(ids[:, c:c + 1] == iota_n,
                                    wcoef[:, c:c + 1], 0.0)
        mean_emb = jnp.dot(wsel, table_ref[...],
                           preferred_element_type=jnp.float32)
        fused = (jnp.dot(mean_emb, w_ref[...],
                         preferred_element_type=jnp.float32) + bias_ref[...])

        for cp in gathers:
            cp.wait()
        rows_ref[...] = rows_ref[...] + fused

        scatters = []

        def scatter_row(k):
            i0 = k * CH // L
            i1 = (k + 1) * CH // L
            for i in range(i0, i1):
                g = i * L + pos_ref[i]
                cp = pltpu.make_async_copy(rows_ref.at[i], out_ref.at[g],
                                           row_sem)
                cp.start()
                scatters.append(cp)

        for k in range(nch):
            if k >= LAG:
                out_cp(k - LAG).wait()
                scatter_row(k - LAG)
                nxt = k - LAG + NBUF
                if nxt < nch:
                    in_cp(nxt).start()
            in_cp(k).wait()
            out_cp(k).start()
        for k in range(max(0, nch - LAG), nch):
            out_cp(k).wait()
            scatter_row(k)
        for cp in scatters:
            cp.wait()

    return _body, rows_n


def _piece(nb):
    body, rows_n = _make_body(nb)

    def run(hs_piece, pos, ids, msk, table, W, bias):
        return pl.pallas_call(
            body,
            out_shape=jax.ShapeDtypeStruct((rows_n, H), jnp.float32),
            in_specs=[
                pl.BlockSpec(memory_space=pltpu.HBM),
                pl.BlockSpec(memory_space=pltpu.SMEM),
                pl.BlockSpec(memory_space=pltpu.VMEM),
                pl.BlockSpec(memory_space=pltpu.VMEM),
                pl.BlockSpec(memory_space=pltpu.VMEM),
                pl.BlockSpec(memory_space=pltpu.VMEM),
                pl.BlockSpec(memory_space=pltpu.VMEM),
            ],
            out_specs=pl.BlockSpec(memory_space=pltpu.HBM),
            scratch_shapes=[
                pltpu.VMEM((NBUF, CH, H), jnp.float32),
                pltpu.VMEM((nb, H), jnp.float32),
                pltpu.SemaphoreType.DMA((NBUF,)),
                pltpu.SemaphoreType.DMA((NBUF,)),
                pltpu.SemaphoreType.DMA,
            ],
        )(hs_piece, pos, ids, msk, table, W, bias)

    return run


def kernel(hidden_states, comp_ids, comp_mask, positions, comp_table, W, b):
    pos = positions.astype(jnp.int32)
    ids = comp_ids.astype(jnp.int32)
    bias = b.reshape(1, H)
    hs = hidden_states.reshape(B * L, H)
    run = _piece(B)
    out = run(hs, pos, ids, comp_mask, comp_table, W, bias)
    return out.reshape(B, L, H)


# speedup vs baseline: 1.0149x; 1.0149x over previous
import jax
import jax.numpy as jnp
from jax.experimental import pallas as pl
from jax.experimental.pallas import tpu as pltpu

B, L, H = 16, 2048, 1536
E = 256
C = 3
NC = 26

CH = 2048
NBUF = 4
LAG = 2
WSUB = 4
WCH = CH // WSUB


def _make_body(nb):
    rows_n = nb * L
    nch = rows_n // CH

    def _body(hid_ref, pos_ref, ids_ref, msk_ref, table_ref, w_ref, bias_ref,
              out_ref, buf_ref, rows_ref, in_sems, out_sems, row_sem):
        def in_cp(k):
            return pltpu.make_async_copy(
                hid_ref.at[pl.ds(k * CH, CH)], buf_ref.at[k % NBUF],
                in_sems.at[k % NBUF])

        def out_cp(k, t):
            return pltpu.make_async_copy(
                buf_ref.at[k % NBUF, pl.ds(t * WCH, WCH)],
                out_ref.at[pl.ds(k * CH + t * WCH, WCH)],
                out_sems.at[(k % 2) * WSUB + t])

        for k in range(NBUF):
            in_cp(k).start()

        gathers = []
        for i in range(nb):
            g = i * L + pos_ref[i]
            cp = pltpu.make_async_copy(hid_ref.at[g], rows_ref.at[i], row_sem)
            cp.start()
            gathers.append(cp)

        ids = ids_ref[...]
        msk = msk_ref[...]
        denom = jnp.maximum(jnp.sum(msk, axis=1, keepdims=True), 1.0)
        wcoef = msk / denom
        iota_n = jax.lax.broadcasted_iota(jnp.int32, (nb, NC), 1)
        wsel = jnp.zeros((nb, NC), jnp.float32)
        for c in range(C):
            wsel = wsel + jnp.where(ids[:, c:c + 1] == iota_n,
                                    wcoef[:, c:c + 1], 0.0)
        mean_emb = jnp.dot(wsel, table_ref[...],
                           preferred_element_type=jnp.float32)
        fused = (jnp.dot(mean_emb, w_ref[...],
                         preferred_element_type=jnp.float32) + bias_ref[...])

        for cp in gathers:
            cp.wait()
        rows_ref[...] = rows_ref[...] + fused

        for k in range(nch):
            if k >= LAG:
                for t in range(WSUB):
                    out_cp(k - LAG, t).wait()
                nxt = k - LAG + NBUF
                if nxt < nch:
                    in_cp(nxt).start()
            in_cp(k).wait()
            for t in range(WSUB):
                out_cp(k, t).start()
        for k in range(max(0, nch - LAG), nch):
            for t in range(WSUB):
                out_cp(k, t).wait()

        scatters = []
        for i in range(nb):
            g = i * L + pos_ref[i]
            cp = pltpu.make_async_copy(rows_ref.at[i], out_ref.at[g], row_sem)
            cp.start()
            scatters.append(cp)
        for cp in scatters:
            cp.wait()

    return _body, rows_n


def _piece(nb):
    body, rows_n = _make_body(nb)

    def run(hs_piece, pos, ids, msk, table, W, bias):
        return pl.pallas_call(
            body,
            out_shape=jax.ShapeDtypeStruct((rows_n, H), jnp.float32),
            in_specs=[
                pl.BlockSpec(memory_space=pltpu.HBM),
                pl.BlockSpec(memory_space=pltpu.SMEM),
                pl.BlockSpec(memory_space=pltpu.VMEM),
                pl.BlockSpec(memory_space=pltpu.VMEM),
                pl.BlockSpec(memory_space=pltpu.VMEM),
                pl.BlockSpec(memory_space=pltpu.VMEM),
                pl.BlockSpec(memory_space=pltpu.VMEM),
            ],
            out_specs=pl.BlockSpec(memory_space=pltpu.HBM),
            scratch_shapes=[
                pltpu.VMEM((NBUF, CH, H), jnp.float32),
                pltpu.VMEM((nb, H), jnp.float32),
                pltpu.SemaphoreType.DMA((NBUF,)),
                pltpu.SemaphoreType.DMA((2 * WSUB,)),
                pltpu.SemaphoreType.DMA,
            ],
        )(hs_piece, pos, ids, msk, table, W, bias)

    return run


def kernel(hidden_states, comp_ids, comp_mask, positions, comp_table, W, b):
    pos = positions.astype(jnp.int32)
    ids = comp_ids.astype(jnp.int32)
    bias = b.reshape(1, H)
    hs = hidden_states.reshape(B * L, H)
    run = _piece(B)
    out = run(hs, pos, ids, comp_mask, comp_table, W, bias)
    return out.reshape(B, L, H)
